# Initial kernel scaffold; baseline (speedup 1.0000x reference)
import jax
import jax.numpy as jnp
from jax.experimental import pallas as pl


def kernel(x1, x2, past_targets, Wl1, bl1, Wl2, bl2, Ws1, bs1, Ws2, bs2):
    raise NotImplementedError("write your pallas kernel here")


# speedup vs baseline: 2.3024x; 2.3024x over previous
import jax
import jax.numpy as jnp
from jax.experimental import pallas as pl

_K = 256
_LANES = 128
_TREND_THRESHOLD = 0.55


def _select_body(x1_ref, x2_ref, trend_ref, idx_ref, zval_ref):
    x1 = x1_ref[...]
    x2 = x2_ref[...]
    r, n = x1.shape
    mean = jnp.mean(x1, axis=1, keepdims=True)
    trend = mean > _TREND_THRESHOLD
    z = jnp.where(trend, -x2, x1)
    iota_n = jax.lax.broadcasted_iota(jnp.int32, (r, n), 1)
    iota_k = jax.lax.broadcasted_iota(jnp.int32, (r, _K), 1)

    def body(t, carry):
        zc, vals, idxs = carry
        m = jnp.min(zc, axis=1, keepdims=True)
        eq = zc == m
        a = jnp.min(jnp.where(eq, iota_n, n), axis=1, keepdims=True)
        sel = iota_k == t
        vals = jnp.where(sel, m, vals)
        idxs = jnp.where(sel, a, idxs)
        zc = jnp.where(iota_n == a, jnp.inf, zc)
        return zc, vals, idxs

    vals0 = jnp.zeros((r, _K), jnp.float32)
    idxs0 = jnp.zeros((r, _K), jnp.int32)
    _, vals, idxs = jax.lax.fori_loop(0, _K, body, (z, vals0, idxs0))
    trend_ref[...] = trend.astype(jnp.int32)
    idx_ref[...] = idxs
    zval_ref[...] = vals


def _gather_mlp_body(pt_ref, idx_ref, xval_ref, w1pt_ref, w1x_ref, b1_ref,
                     w2_ref, b2_ref, out_ref):
    ptc = pt_ref[0]
    c, d = ptc.shape
    idxv = idx_ref[0]
    hi = idxv // _LANES
    lo = idxv - hi * _LANES
    iota_c = jax.lax.broadcasted_iota(jnp.int32, (_K, c), 1)
    oh_hi = (iota_c == hi).astype(jnp.float32)
    g = jnp.dot(oh_hi, ptc, preferred_element_type=jnp.float32)
    iota_d = jax.lax.broadcasted_iota(jnp.int32, (_K, d), 1)
    oh_lo = ((iota_d % _LANES) == lo).astype(jnp.float32)
    prod = g * oh_lo
    nf = d // _LANES
    iota_dr = jax.lax.broadcasted_iota(jnp.int32, (d, 8), 0)
    iota_fc = jax.lax.broadcasted_iota(jnp.int32, (d, 8), 1)
    s = ((iota_dr // _LANES) == iota_fc).astype(jnp.float32)
    feats = jnp.dot(prod, s, preferred_element_type=jnp.float32)
    del nf
    xv = xval_ref[0]
    w1pt = w1pt_ref[0]
    w1x = w1x_ref[0]
    b1 = b1_ref[0]
    w2 = w2_ref[0]
    b2 = b2_ref[0][:, 0:1]
    h = jnp.dot(feats, w1pt, preferred_element_type=jnp.float32)
    h = h + xv * w1x + b1
    h = jnp.maximum(h, 0.0)
    o = jnp.sum(h * w2, axis=1, keepdims=True) + b2
    out_ref[0] = o


def kernel(x1, x2, past_targets, Wl1, bl1, Wl2, bl2, Ws1, bs1, Ws2, bs2):
    b, n = x1.shape
    nf = past_targets.shape[-1]
    h = Wl1.shape[1]
    c = n // _LANES
    rb = min(16, b)

    trend_i, idx, zval = pl.pallas_call(
        _select_body,
        grid=(b // rb,),
        in_specs=[
            pl.BlockSpec((rb, n), lambda i: (i, 0)),
            pl.BlockSpec((rb, n), lambda i: (i, 0)),
        ],
        out_specs=[
            pl.BlockSpec((rb, 1), lambda i: (i, 0)),
            pl.BlockSpec((rb, _K), lambda i: (i, 0)),
            pl.BlockSpec((rb, _K), lambda i: (i, 0)),
        ],
        out_shape=[
            jax.ShapeDtypeStruct((b, 1), jnp.int32),
            jax.ShapeDtypeStruct((b, _K), jnp.int32),
            jax.ShapeDtypeStruct((b, _K), jnp.float32),
        ],
    )(x1, x2)

    trend = trend_i[:, 0] > 0
    xval = jnp.where(trend[:, None], -zval, zval).reshape(b, _K, 1)
    idx3 = idx.reshape(b, _K, 1)

    pt2 = (past_targets
           .reshape(b, c, _LANES, nf)
           .transpose(0, 1, 3, 2)
           .reshape(b, c, nf * _LANES))

    tsel = trend[:, None, None]
    w1 = jnp.where(tsel, Wl1[None], Ws1[None])
    w1x = w1[:, 0:1, :]
    w1pt = jnp.pad(w1[:, 1:, :], ((0, 0), (0, 8 - (w1.shape[1] - 1)), (0, 0)))
    b1 = jnp.where(tsel, bl1[None, None, :], bs1[None, None, :])
    w2 = jnp.where(tsel, Wl2.T[None], Ws2.T[None])
    b2 = jnp.where(tsel, bl2[None, None, :], bs2[None, None, :])
    b2 = jnp.broadcast_to(b2, (b, 1, h)) * jnp.ones((1, 1, h), jnp.float32)

    out3 = pl.pallas_call(
        _gather_mlp_body,
        grid=(b,),
        in_specs=[
            pl.BlockSpec((1, c, nf * _LANES), lambda r: (r, 0, 0)),
            pl.BlockSpec((1, _K, 1), lambda r: (r, 0, 0)),
            pl.BlockSpec((1, _K, 1), lambda r: (r, 0, 0)),
            pl.BlockSpec((1, 8, h), lambda r: (r, 0, 0)),
            pl.BlockSpec((1, 1, h), lambda r: (r, 0, 0)),
            pl.BlockSpec((1, 1, h), lambda r: (r, 0, 0)),
            pl.BlockSpec((1, 1, h), lambda r: (r, 0, 0)),
            pl.BlockSpec((1, 1, h), lambda r: (r, 0, 0)),
        ],
        out_specs=pl.BlockSpec((1, _K, 1), lambda r: (r, 0, 0)),
        out_shape=jax.ShapeDtypeStruct((b, _K, 1), jnp.float32),
    )(pt2, idx3, xval, w1pt, w1x, b1, w2, b2)

    return out3.reshape(b, _K), trend, idx


# speedup vs baseline: 2.5411x; 1.1037x over previous
import jax
import jax.numpy as jnp
from jax.experimental import pallas as pl

_K = 256
_LANES = 128
_TREND_THRESHOLD = 0.55


def _select_body(x1_ref, x2_ref, trend_ref, idx_ref, zval_ref):
    x1 = x1_ref[...]
    x2 = x2_ref[...]
    rc, l = x1.shape
    r = _K // 16
    c = rc // r
    iota_j = jax.lax.broadcasted_iota(jnp.int32, (r, rc), 1)
    iota_r = jax.lax.broadcasted_iota(jnp.int32, (r, 1), 0)
    iota_c = jax.lax.broadcasted_iota(jnp.int32, (r, c), 1)
    iota_l = jax.lax.broadcasted_iota(jnp.int32, (r, l), 1)
    iota_k = jax.lax.broadcasted_iota(jnp.int32, (r, _K), 1)
    inf = jnp.float32(jnp.inf)

    sel = (iota_j // c == iota_r).astype(jnp.float32)
    s1 = jnp.dot(sel, x1, preferred_element_type=jnp.float32)
    mean = jnp.sum(s1, axis=1, keepdims=True) / jnp.float32(rc // r * l)
    trend = mean > _TREND_THRESHOLD
    selt = (jax.lax.broadcasted_iota(jnp.int32, (rc, r), 0) // c ==
            jax.lax.broadcasted_iota(jnp.int32, (rc, r), 1)).astype(
                jnp.float32)
    tbig = jnp.dot(selt, jnp.broadcast_to(
        trend.astype(jnp.float32), (r, l)),
        preferred_element_type=jnp.float32)
    z = jnp.where(tbig > 0.5, -x2, x1)

    h1 = z.astype(jnp.bfloat16)
    r1 = z - h1.astype(jnp.float32)
    h2 = r1.astype(jnp.bfloat16)
    h3 = (r1 - h2.astype(jnp.float32)).astype(jnp.bfloat16)
    selb = sel.astype(jnp.bfloat16)

    def exact_sel(oh_b, p1, p2, p3):
        d1 = jnp.dot(oh_b, p1, preferred_element_type=jnp.float32)
        d2 = jnp.dot(oh_b, p2, preferred_element_type=jnp.float32)
        d3 = jnp.dot(oh_b, p3, preferred_element_type=jnp.float32)
        return (d1 + d2) + d3

    minl = jnp.min(z, axis=1, keepdims=True)
    ohcol = ((jax.lax.broadcasted_iota(jnp.int32, (rc, c), 0) % c) ==
             jax.lax.broadcasted_iota(jnp.int32, (rc, c), 1)).astype(
                 jnp.float32)
    m1 = minl.astype(jnp.bfloat16)
    mr1 = minl - m1.astype(jnp.float32)
    m2 = mr1.astype(jnp.bfloat16)
    m3 = (mr1 - m2.astype(jnp.float32)).astype(jnp.bfloat16)
    cm = exact_sel(
        selb,
        (jnp.broadcast_to(m1.astype(jnp.float32), (rc, c)) * ohcol
         ).astype(jnp.bfloat16),
        (jnp.broadcast_to(m2.astype(jnp.float32), (rc, c)) * ohcol
         ).astype(jnp.bfloat16),
        (jnp.broadcast_to(m3.astype(jnp.float32), (rc, c)) * ohcol
         ).astype(jnp.bfloat16))

    wv = jnp.full((r, c), -inf)
    wl = jnp.full((r, c), -1, jnp.int32)

    def body(t, carry):
        cm, wv, wl, vals, idxs = carry
        gm = jnp.min(cm, axis=1, keepdims=True)
        cidx = jnp.min(jnp.where(cm == gm, iota_c, c), axis=1,
                       keepdims=True)
        ohc = iota_c == cidx
        oh = (iota_j == iota_r * c + cidx).astype(jnp.bfloat16)
        zc = exact_sel(oh, h1, h2, h3)
        wvc = jnp.max(jnp.where(ohc, wv, -inf), axis=1, keepdims=True)
        wlc = jnp.max(jnp.where(ohc, wl, -1), axis=1, keepdims=True)
        done = (zc < wvc) | ((zc == wvc) & (iota_l <= wlc))
        zm = jnp.where(done, inf, zc)
        lane = jnp.min(jnp.where(zm == gm, iota_l, l), axis=1,
                       keepdims=True)
        newcm = jnp.min(
            jnp.where(done | (zm < gm) | ((zm == gm) & (iota_l <= lane)),
                      inf, zc), axis=1, keepdims=True)
        cm = jnp.where(ohc, newcm, cm)
        wv = jnp.where(ohc, gm, wv)
        wl = jnp.where(ohc, lane, wl)
        sel_t = iota_k == t
        vals = jnp.where(sel_t, gm, vals)
        idxs = jnp.where(sel_t, cidx * l + lane, idxs)
        return cm, wv, wl, vals, idxs

    vals0 = jnp.zeros((r, _K), jnp.float32)
    idxs0 = jnp.zeros((r, _K), jnp.int32)
    _, _, _, vals, idxs = jax.lax.fori_loop(
        0, _K, body, (cm, wv, wl, vals0, idxs0))
    trend_ref[...] = trend.astype(jnp.int32)
    idx_ref[...] = idxs
    zval_ref[...] = vals


def _gather_mlp_body(pt_ref, idx_ref, xval_ref, w1pt_ref, w1x_ref, b1_ref,
                     w2_ref, b2_ref, out_ref):
    ptc = pt_ref[0]
    c, d = ptc.shape
    idxv = idx_ref[0]
    hi = idxv // _LANES
    lo = idxv - hi * _LANES
    iota_c = jax.lax.broadcasted_iota(jnp.int32, (_K, c), 1)
    oh_hi = (iota_c == hi).astype(jnp.float32)
    g = jnp.dot(oh_hi, ptc, preferred_element_type=jnp.float32)
    iota_d = jax.lax.broadcasted_iota(jnp.int32, (_K, d), 1)
    oh_lo = ((iota_d % _LANES) == lo).astype(jnp.float32)
    prod = g * oh_lo
    iota_dr = jax.lax.broadcasted_iota(jnp.int32, (d, 8), 0)
    iota_fc = jax.lax.broadcasted_iota(jnp.int32, (d, 8), 1)
    s = ((iota_dr // _LANES) == iota_fc).astype(jnp.float32)
    feats = jnp.dot(prod, s, preferred_element_type=jnp.float32)
    xv = xval_ref[0]
    w1pt = w1pt_ref[0]
    w1x = w1x_ref[0]
    b1 = b1_ref[0]
    w2 = w2_ref[0]
    b2 = b2_ref[0][:, 0:1]
    h = jnp.dot(feats, w1pt, preferred_element_type=jnp.float32)
    h = h + xv * w1x + b1
    h = jnp.maximum(h, 0.0)
    o = jnp.sum(h * w2, axis=1, keepdims=True) + b2
    out_ref[0] = o


def kernel(x1, x2, past_targets, Wl1, bl1, Wl2, bl2, Ws1, bs1, Ws2, bs2):
    b, n = x1.shape
    nf = past_targets.shape[-1]
    h = Wl1.shape[1]
    c = n // _LANES
    rb = min(16, b)

    x1c = x1.reshape(b * c, _LANES)
    x2c = x2.reshape(b * c, _LANES)

    trend_i, idx, zval = pl.pallas_call(
        _select_body,
        grid=(b // rb,),
        in_specs=[
            pl.BlockSpec((rb * c, _LANES), lambda i: (i, 0)),
            pl.BlockSpec((rb * c, _LANES), lambda i: (i, 0)),
        ],
        out_specs=[
            pl.BlockSpec((rb, 1), lambda i: (i, 0)),
            pl.BlockSpec((rb, _K), lambda i: (i, 0)),
            pl.BlockSpec((rb, _K), lambda i: (i, 0)),
        ],
        out_shape=[
            jax.ShapeDtypeStruct((b, 1), jnp.int32),
            jax.ShapeDtypeStruct((b, _K), jnp.int32),
            jax.ShapeDtypeStruct((b, _K), jnp.float32),
        ],
    )(x1c, x2c)

    trend = trend_i[:, 0] > 0
    xval = jnp.where(trend[:, None], -zval, zval).reshape(b, _K, 1)
    idx3 = idx.reshape(b, _K, 1)

    pt2 = (past_targets
           .reshape(b, c, _LANES, nf)
           .transpose(0, 1, 3, 2)
           .reshape(b, c, nf * _LANES))

    tsel = trend[:, None, None]
    w1 = jnp.where(tsel, Wl1[None], Ws1[None])
    w1x = w1[:, 0:1, :]
    w1pt = jnp.pad(w1[:, 1:, :], ((0, 0), (0, 8 - (w1.shape[1] - 1)), (0, 0)))
    b1 = jnp.where(tsel, bl1[None, None, :], bs1[None, None, :])
    w2 = jnp.where(tsel, Wl2.T[None], Ws2.T[None])
    b2 = jnp.where(tsel, bl2[None, None, :], bs2[None, None, :])
    b2 = jnp.broadcast_to(b2, (b, 1, h)) * jnp.ones((1, 1, h), jnp.float32)

    out3 = pl.pallas_call(
        _gather_mlp_body,
        grid=(b,),
        in_specs=[
            pl.BlockSpec((1, c, nf * _LANES), lambda r: (r, 0, 0)),
            pl.BlockSpec((1, _K, 1), lambda r: (r, 0, 0)),
            pl.BlockSpec((1, _K, 1), lambda r: (r, 0, 0)),
            pl.BlockSpec((1, 8, h), lambda r: (r, 0, 0)),
            pl.BlockSpec((1, 1, h), lambda r: (r, 0, 0)),
            pl.BlockSpec((1, 1, h), lambda r: (r, 0, 0)),
            pl.BlockSpec((1, 1, h), lambda r: (r, 0, 0)),
            pl.BlockSpec((1, 1, h), lambda r: (r, 0, 0)),
        ],
        out_specs=pl.BlockSpec((1, _K, 1), lambda r: (r, 0, 0)),
        out_shape=jax.ShapeDtypeStruct((b, _K, 1), jnp.float32),
    )(pt2, idx3, xval, w1pt, w1x, b1, w2, b2)

    return out3.reshape(b, _K), trend, idx
